# baseline (speedup 1.0000x reference)
import functools

import jax
import jax.numpy as jnp
from jax.experimental import pallas as pl
from jax.experimental.pallas import tpu as pltpu


def _round_up(x, m):
    return (x + m - 1) // m * m


def _row(table_ref, base_ref, rem_ref, r):
    base = pl.multiple_of(base_ref[r], 8)
    chunk = table_ref[pl.ds(base, 8), :]
    return pltpu.roll(chunk, rem_ref[r], axis=0)[0:1, :]


def _gather_mul_body(cb_ref, cr_ref, lb_ref, lr_ref, table_ref, o_ref, *,
                     rows_per_group):
    tb = o_ref.shape[0]
    roff = pl.program_id(1) * tb

    def group(g, carry):
        lbase = pl.multiple_of(g * rows_per_group, 8)
        rows = []
        for j in range(rows_per_group):
            r = roff + lbase + j
            rows.append(
                _row(table_ref, cb_ref, cr_ref, r)
                * _row(table_ref, lb_ref, lr_ref, r)
            )
        for k in range(rows_per_group // 8):
            obase = pl.multiple_of(lbase + k * 8, 8)
            o_ref[pl.ds(obase, 8), :] = jnp.concatenate(rows[k * 8:(k + 1) * 8], axis=0)
        return carry

    jax.lax.fori_loop(0, tb // rows_per_group, group, 0)


def _gather_body(cb_ref, cr_ref, table_ref, o_ref, *, rows_per_group):
    tb = o_ref.shape[0]
    roff = pl.program_id(1) * tb

    def group(g, carry):
        lbase = pl.multiple_of(g * rows_per_group, 8)
        rows = [_row(table_ref, cb_ref, cr_ref, roff + lbase + j)
                for j in range(rows_per_group)]
        for k in range(rows_per_group // 8):
            obase = pl.multiple_of(lbase + k * 8, 8)
            o_ref[pl.ds(obase, 8), :] = jnp.concatenate(rows[k * 8:(k + 1) * 8], axis=0)
        return carry

    jax.lax.fori_loop(0, tb // rows_per_group, group, 0)


def kernel(emb_table, current_ids, last_ids=None):
    V, D = emb_table.shape
    B = current_ids.shape[0]

    NC = 2 if (D % 256 == 0) else 1
    Dc = D // NC

    ROWS_PER_GROUP = 128
    NB = 4
    B_pad = _round_up(max(B, 1), ROWS_PER_GROUP * NB)
    TB = B_pad // NB

    def prep_ids(ids):
        ids = jnp.clip(ids.astype(jnp.int32), 0, V - 1)
        ids = jnp.pad(ids, (0, B_pad - B))
        return ids & ~7, (8 - (ids & 7)) & 7

    table_spec = pl.BlockSpec((V, Dc), lambda i, j, *_: (0, i))
    out_spec = pl.BlockSpec((TB, Dc), lambda i, j, *_: (j, i))
    out_shape = jax.ShapeDtypeStruct((B_pad, D), emb_table.dtype)

    itemsize = jnp.dtype(emb_table.dtype).itemsize
    n_ids = 1 if last_ids is None else 2
    cost = pl.CostEstimate(
        flops=n_ids * B_pad * D,
        transcendentals=0,
        bytes_accessed=V * D * itemsize + B_pad * D * itemsize + n_ids * B_pad * 4,
    )
    compiler_params = pltpu.CompilerParams(
        dimension_semantics=("parallel", "arbitrary"),
    )

    if last_ids is None:
        body = functools.partial(_gather_body, rows_per_group=ROWS_PER_GROUP)
        cb, cr = prep_ids(current_ids)
        out = pl.pallas_call(
            body,
            out_shape=out_shape,
            grid_spec=pltpu.PrefetchScalarGridSpec(
                num_scalar_prefetch=2,
                grid=(NC, NB),
                in_specs=[table_spec],
                out_specs=out_spec,
            ),
            compiler_params=compiler_params,
            cost_estimate=cost,
        )(cb, cr, emb_table)
    else:
        body = functools.partial(_gather_mul_body, rows_per_group=ROWS_PER_GROUP)
        cb, cr = prep_ids(current_ids)
        lb, lr = prep_ids(last_ids)
        out = pl.pallas_call(
            body,
            out_shape=out_shape,
            grid_spec=pltpu.PrefetchScalarGridSpec(
                num_scalar_prefetch=4,
                grid=(NC, NB),
                in_specs=[table_spec],
                out_specs=out_spec,
            ),
            compiler_params=compiler_params,
            cost_estimate=cost,
        )(cb, cr, lb, lr, emb_table)

    return out[:B]


# speedup vs baseline: 1.1392x; 1.1392x over previous
import functools

import jax
import jax.numpy as jnp
from jax.experimental import pallas as pl
from jax.experimental.pallas import tpu as pltpu


def _round_up(x, m):
    return (x + m - 1) // m * m


def _row(table_ref, base_ref, rem_ref, r):
    base = pl.multiple_of(base_ref[r], 8)
    chunk = table_ref[pl.ds(base, 8), :]
    return pltpu.roll(chunk, rem_ref[r], axis=0)[0:1, :]


def _gather_mul_body(cb_ref, cr_ref, lb_ref, lr_ref, table_ref, o_ref, *,
                     groups, rows_per_group):
    def group(g, carry):
        gbase = pl.multiple_of(g * rows_per_group, 8)
        for k in range(rows_per_group // 8):
            obase = pl.multiple_of(gbase + k * 8, 8)
            rows = []
            for j in range(8):
                r = obase + j
                rows.append(
                    _row(table_ref, cb_ref, cr_ref, r)
                    * _row(table_ref, lb_ref, lr_ref, r)
                )
            o_ref[pl.ds(obase, 8), :] = jnp.concatenate(rows, axis=0)
        return carry

    jax.lax.fori_loop(0, groups, group, 0)


def _gather_body(cb_ref, cr_ref, table_ref, o_ref, *, groups, rows_per_group):
    def group(g, carry):
        gbase = pl.multiple_of(g * rows_per_group, 8)
        for k in range(rows_per_group // 8):
            obase = pl.multiple_of(gbase + k * 8, 8)
            rows = [_row(table_ref, cb_ref, cr_ref, obase + j) for j in range(8)]
            o_ref[pl.ds(obase, 8), :] = jnp.concatenate(rows, axis=0)
        return carry

    jax.lax.fori_loop(0, groups, group, 0)


def kernel(emb_table, current_ids, last_ids=None):
    V, D = emb_table.shape
    B = current_ids.shape[0]

    NC = 2 if (D % 256 == 0) else 1
    Dc = D // NC

    ROWS_PER_GROUP = 128
    B_pad = _round_up(max(B, 1), ROWS_PER_GROUP)
    groups = B_pad // ROWS_PER_GROUP

    def prep_ids(ids):
        ids = jnp.clip(ids.astype(jnp.int32), 0, V - 1)
        ids = jnp.pad(ids, (0, B_pad - B))
        return ids & ~7, (8 - (ids & 7)) & 7

    table_spec = pl.BlockSpec((V, Dc), lambda i, *_: (0, i))
    out_spec = pl.BlockSpec((B_pad, Dc), lambda i, *_: (0, i))
    out_shape = jax.ShapeDtypeStruct((B_pad, D), emb_table.dtype)

    itemsize = jnp.dtype(emb_table.dtype).itemsize
    n_ids = 1 if last_ids is None else 2
    cost = pl.CostEstimate(
        flops=n_ids * B_pad * D,
        transcendentals=0,
        bytes_accessed=V * D * itemsize + B_pad * D * itemsize + n_ids * B_pad * 4,
    )
    compiler_params = pltpu.CompilerParams(dimension_semantics=("parallel",))

    if last_ids is None:
        body = functools.partial(
            _gather_body, groups=groups, rows_per_group=ROWS_PER_GROUP
        )
        cb, cr = prep_ids(current_ids)
        out = pl.pallas_call(
            body,
            out_shape=out_shape,
            grid_spec=pltpu.PrefetchScalarGridSpec(
                num_scalar_prefetch=2,
                grid=(NC,),
                in_specs=[table_spec],
                out_specs=out_spec,
            ),
            compiler_params=compiler_params,
            cost_estimate=cost,
        )(cb, cr, emb_table)
    else:
        body = functools.partial(
            _gather_mul_body, groups=groups, rows_per_group=ROWS_PER_GROUP
        )
        cb, cr = prep_ids(current_ids)
        lb, lr = prep_ids(last_ids)
        out = pl.pallas_call(
            body,
            out_shape=out_shape,
            grid_spec=pltpu.PrefetchScalarGridSpec(
                num_scalar_prefetch=4,
                grid=(NC,),
                in_specs=[table_spec],
                out_specs=out_spec,
            ),
            compiler_params=compiler_params,
            cost_estimate=cost,
        )(cb, cr, lb, lr, emb_table)

    return out[:B]


# speedup vs baseline: 1.2510x; 1.0982x over previous
import functools

import jax
import jax.numpy as jnp
from jax.experimental import pallas as pl
from jax.experimental.pallas import tpu as pltpu


def _round_up(x, m):
    return (x + m - 1) // m * m


def _row(table_ref, base_ref, rem_ref, r):
    base = pl.multiple_of(base_ref[r], 8)
    chunk = table_ref[pl.ds(base, 8), :]
    return pltpu.roll(chunk, rem_ref[r], axis=0)[0:1, :]


def _gather_mul_body(cb_ref, cr_ref, lb_ref, lr_ref, table_ref, o_ref, *,
                     groups, rows_per_group):
    def group(g, carry):
        gbase = pl.multiple_of(g * rows_per_group, 8)
        rows = []
        for j in range(rows_per_group):
            r = gbase + j
            rows.append(
                _row(table_ref, cb_ref, cr_ref, r)
                * _row(table_ref, lb_ref, lr_ref, r)
            )
        for k in range(rows_per_group // 8):
            obase = pl.multiple_of(gbase + k * 8, 8)
            o_ref[pl.ds(obase, 8), :] = jnp.concatenate(rows[k * 8:(k + 1) * 8], axis=0)
        return carry

    jax.lax.fori_loop(0, groups, group, 0)


def _gather_body(cb_ref, cr_ref, table_ref, o_ref, *, groups, rows_per_group):
    def group(g, carry):
        gbase = pl.multiple_of(g * rows_per_group, 8)
        rows = [_row(table_ref, cb_ref, cr_ref, gbase + j)
                for j in range(rows_per_group)]
        for k in range(rows_per_group // 8):
            obase = pl.multiple_of(gbase + k * 8, 8)
            o_ref[pl.ds(obase, 8), :] = jnp.concatenate(rows[k * 8:(k + 1) * 8], axis=0)
        return carry

    jax.lax.fori_loop(0, groups, group, 0)


def kernel(emb_table, current_ids, last_ids=None):
    V, D = emb_table.shape
    B = current_ids.shape[0]

    NC = 2 if (D % 256 == 0) else 1
    Dc = D // NC

    ROWS_PER_GROUP = 256
    B_pad = _round_up(max(B, 1), ROWS_PER_GROUP)
    groups = B_pad // ROWS_PER_GROUP

    def prep_ids(ids):
        ids = jnp.clip(ids.astype(jnp.int32), 0, V - 1)
        ids = jnp.pad(ids, (0, B_pad - B))
        return ids & ~7, (8 - (ids & 7)) & 7

    table_spec = pl.BlockSpec((V, Dc), lambda i, *_: (0, i))
    out_spec = pl.BlockSpec((B_pad, Dc), lambda i, *_: (0, i))
    out_shape = jax.ShapeDtypeStruct((B_pad, D), emb_table.dtype)

    itemsize = jnp.dtype(emb_table.dtype).itemsize
    n_ids = 1 if last_ids is None else 2
    cost = pl.CostEstimate(
        flops=n_ids * B_pad * D,
        transcendentals=0,
        bytes_accessed=V * D * itemsize + B_pad * D * itemsize + n_ids * B_pad * 4,
    )
    compiler_params = pltpu.CompilerParams(dimension_semantics=("parallel",))

    if last_ids is None:
        body = functools.partial(
            _gather_body, groups=groups, rows_per_group=ROWS_PER_GROUP
        )
        cb, cr = prep_ids(current_ids)
        out = pl.pallas_call(
            body,
            out_shape=out_shape,
            grid_spec=pltpu.PrefetchScalarGridSpec(
                num_scalar_prefetch=2,
                grid=(NC,),
                in_specs=[table_spec],
                out_specs=out_spec,
            ),
            compiler_params=compiler_params,
            cost_estimate=cost,
        )(cb, cr, emb_table)
    else:
        body = functools.partial(
            _gather_mul_body, groups=groups, rows_per_group=ROWS_PER_GROUP
        )
        cb, cr = prep_ids(current_ids)
        lb, lr = prep_ids(last_ids)
        out = pl.pallas_call(
            body,
            out_shape=out_shape,
            grid_spec=pltpu.PrefetchScalarGridSpec(
                num_scalar_prefetch=4,
                grid=(NC,),
                in_specs=[table_spec],
                out_specs=out_spec,
            ),
            compiler_params=compiler_params,
            cost_estimate=cost,
        )(cb, cr, lb, lr, emb_table)

    return out[:B]
